# Initial kernel scaffold; baseline (speedup 1.0000x reference)
import jax
import jax.numpy as jnp
from jax.experimental import pallas as pl


def kernel(x, edge_index, edge_attr, batch, token_index, W, bias):
    raise NotImplementedError("write your pallas kernel here")


# speedup vs baseline: 24.3594x; 24.3594x over previous
import jax
import jax.numpy as jnp
from jax import lax
from jax.experimental import pallas as pl
from jax.experimental.pallas import tpu as pltpu
from jax.experimental.pallas import tpu_sc as plsc

N = 10240
E = 163840
D = 128
NG = 256
NPG = 40
R = NG * NPG * NPG

SHIFT = 13
BUCKET = 1 << SHIFT
NBUCKET = R // BUCKET
NCORE = 2
NSUB = 16
NPASS = NBUCKET // NCORE
LANES = 16
EC = E // NSUB
NV = EC // LANES
CH = 128
PAD_ROWS = 8
TPT = BUCKET // NSUB
IDB = 14
IDM = (1 << IDB) - 1


def _mm_body(x_ref, w_ref, b_ref, g0_ref, g1_ref):
    xb = x_ref[...]
    w = w_ref[...]
    g0_ref[...] = (
        jnp.dot(xb, w[:D, :], preferred_element_type=jnp.float32) + b_ref[...]
    )
    g1_ref[...] = jnp.dot(xb, w[D:, :], preferred_element_type=jnp.float32)


def _node_tables(x, w, bias):
    bm = 2048
    return pl.pallas_call(
        _mm_body,
        grid=(N // bm,),
        in_specs=[
            pl.BlockSpec((bm, D), lambda i: (i, 0)),
            pl.BlockSpec((2 * D, D), lambda i: (0, 0)),
            pl.BlockSpec((1, D), lambda i: (0, 0)),
        ],
        out_specs=[
            pl.BlockSpec((bm, D), lambda i: (i, 0)),
            pl.BlockSpec((bm, D), lambda i: (i, 0)),
        ],
        out_shape=[
            jax.ShapeDtypeStruct((N, D), jnp.float32),
            jax.ShapeDtypeStruct((N, D), jnp.float32),
        ],
    )(x, w, bias.reshape(1, D))


def _lane0(v):
    return jnp.max(jnp.where(lax.iota(jnp.int32, LANES) == 0, v, 0))


def _mod40(v):
    return v - ((v * 52429) >> 21) * NPG


def _sc_body(src_h, dst_h, t0_h, t1_h, ea_h, g0_h, g1_h, zsrc_h, out_h,
             fe_v, t01_v, bins_a, bins_b, off_a, off_b, ptr_a, ptr_b,
             gidx, gidx2, ridx, rb, acc, sem):
    c = lax.axis_index("c")
    s = lax.axis_index("s")
    base_e = s * EC
    iota = lax.iota(jnp.int32, LANES)
    zeros16 = jnp.zeros((LANES,), jnp.int32)

    for k in range(NBUCKET + 2):
        off_a[pl.ds(k * LANES, LANES)] = zeros16
        off_b[pl.ds(k * LANES, LANES)] = zeros16

    pltpu.sync_copy(src_h.at[pl.ds(base_e, EC)], fe_v)
    pltpu.sync_copy(dst_h.at[pl.ds(base_e, EC)], bins_a)

    def histo_e(i, _):
        src16 = fe_v[pl.ds(i * LANES, LANES)]
        dst16 = bins_a[pl.ds(i * LANES, LANES)]
        fe16 = src16 * NPG + _mod40(dst16)
        fe_v[pl.ds(i * LANES, LANES)] = fe16
        sla = ((fe16 >> SHIFT) << 4) + iota
        plsc.store_scatter(off_a, [sla], plsc.load_gather(off_a, [sla]) + 1)
        return 0

    lax.fori_loop(0, NV, histo_e, 0)

    pltpu.sync_copy(t0_h.at[pl.ds(base_e, EC)], t01_v)
    pltpu.sync_copy(t1_h.at[pl.ds(base_e, EC)], bins_b)

    def histo_t(i, _):
        t016 = t01_v[pl.ds(i * LANES, LANES)]
        t116 = bins_b[pl.ds(i * LANES, LANES)]
        ft16 = t016 * NPG + _mod40(t116)
        t01_v[pl.ds(i * LANES, LANES)] = t016 + (t116 << IDB)
        slb = ((ft16 >> SHIFT) << 4) + iota
        plsc.store_scatter(off_b, [slb], plsc.load_gather(off_b, [slb]) + 1)
        return 0

    lax.fori_loop(0, NV, histo_t, 0)

    def prefix(off_ref):
        def pbody(k, carry):
            v = off_ref[pl.ds(k * LANES, LANES)]
            incl = plsc.cumsum(v)
            off_ref[pl.ds(k * LANES, LANES)] = incl - v + carry
            return carry + jnp.max(incl)

        lax.fori_loop(0, NBUCKET + 1, pbody, jnp.int32(0))

    prefix(off_a)
    prefix(off_b)
    for k in range(NBUCKET + 1):
        ptr_a[pl.ds(k * LANES, LANES)] = off_a[pl.ds(k * LANES, LANES)]
        ptr_b[pl.ds(k * LANES, LANES)] = off_b[pl.ds(k * LANES, LANES)]

    def permute_e(i, _):
        id16 = i * LANES + iota
        fe16 = fe_v[pl.ds(i * LANES, LANES)]
        sla = ((fe16 >> SHIFT) << 4) + iota
        pa = plsc.load_gather(ptr_a, [sla])
        plsc.store_scatter(
            bins_a, [pa], ((fe16 & (BUCKET - 1)) << IDB) + id16)
        plsc.store_scatter(ptr_a, [sla], pa + 1)
        return 0

    lax.fori_loop(0, NV, permute_e, 0)

    def permute_t(i, _):
        id16 = i * LANES + iota
        t01 = t01_v[pl.ds(i * LANES, LANES)]
        t016 = t01 & IDM
        t116 = t01 >> IDB
        ft16 = t016 * NPG + _mod40(t116)
        slb = ((ft16 >> SHIFT) << 4) + iota
        pb = plsc.load_gather(ptr_b, [slb])
        plsc.store_scatter(
            bins_b, [pb], ((ft16 & (BUCKET - 1)) << IDB) + id16)
        plsc.store_scatter(ptr_b, [slb], pb + 1)
        return 0

    lax.fori_loop(0, NV, permute_t, 0)

    my_row0 = s * TPT
    pad_rows = BUCKET + (iota & (PAD_ROWS - 1))
    pad_nodes = s * LANES + iota

    def pass_body(p, _):
        b = NCORE * p + c
        base_row = b * BUCKET

        pltpu.sync_copy(zsrc_h, acc.at[pl.ds(my_row0, TPT)])
        plsc.subcore_barrier()

        sa = _lane0(off_a[pl.ds(b * LANES, LANES)])
        ea = _lane0(off_a[pl.ds((b + 1) * LANES, LANES)])
        sb = _lane0(off_b[pl.ds(b * LANES, LANES)])
        eb = _lane0(off_b[pl.ds((b + 1) * LANES, LANES)])

        def chunk_a(ci, _):
            st = sa + ci * CH
            for k in range(CH // LANES):
                n16 = st + k * LANES + iota
                valid = n16 < ea
                pk = plsc.load_gather(bins_a, [jnp.where(valid, n16, sa)])
                gidx[pl.ds(k * LANES, LANES)] = jnp.where(
                    valid, (pk & IDM) + base_e, pad_nodes)
                ridx[pl.ds(k * LANES, LANES)] = jnp.where(
                    valid, pk >> IDB, pad_rows)
            pltpu.async_copy(ea_h.at[gidx], rb, sem).wait()
            pltpu.sync_copy(rb, acc.at[ridx], add=True)
            return 0

        lax.fori_loop(0, (ea - sa + CH - 1) >> 7, chunk_a, 0)

        def chunk_b(ci, _):
            st = sb + ci * CH
            for k in range(CH // LANES):
                n16 = st + k * LANES + iota
                valid = n16 < eb
                pk = plsc.load_gather(bins_b, [jnp.where(valid, n16, sb)])
                t01 = plsc.load_gather(t01_v, [pk & IDM])
                gidx[pl.ds(k * LANES, LANES)] = jnp.where(
                    valid, t01 & IDM, pad_nodes)
                gidx2[pl.ds(k * LANES, LANES)] = jnp.where(
                    valid, t01 >> IDB, pad_nodes)
                ridx[pl.ds(k * LANES, LANES)] = jnp.where(
                    valid, pk >> IDB, pad_rows)
            pltpu.async_copy(g0_h.at[gidx], rb, sem).wait()
            pltpu.sync_copy(rb, acc.at[ridx], add=True)
            pltpu.async_copy(g1_h.at[gidx2], rb, sem).wait()
            pltpu.sync_copy(rb, acc.at[ridx], add=True)
            return 0

        lax.fori_loop(0, (eb - sb + CH - 1) >> 7, chunk_b, 0)

        plsc.subcore_barrier()
        pltpu.sync_copy(
            acc.at[pl.ds(my_row0, TPT)],
            out_h.at[pl.ds(base_row + my_row0, TPT)],
        )
        return 0

    lax.fori_loop(0, NPASS, pass_body, 0)


_SC_SCRATCH = [
    pltpu.VMEM((EC,), jnp.int32),
    pltpu.VMEM((EC,), jnp.int32),
    pltpu.VMEM((EC,), jnp.int32),
    pltpu.VMEM((EC,), jnp.int32),
    pltpu.VMEM(((NBUCKET + 2) * LANES,), jnp.int32),
    pltpu.VMEM(((NBUCKET + 2) * LANES,), jnp.int32),
    pltpu.VMEM(((NBUCKET + 2) * LANES,), jnp.int32),
    pltpu.VMEM(((NBUCKET + 2) * LANES,), jnp.int32),
    pltpu.VMEM((CH,), jnp.int32),
    pltpu.VMEM((CH,), jnp.int32),
    pltpu.VMEM((CH,), jnp.int32),
    pltpu.VMEM((CH, D), jnp.float32),
    pltpu.VMEM_SHARED((BUCKET + PAD_ROWS, D), jnp.float32),
    pltpu.SemaphoreType.DMA,
]


@jax.jit
def kernel(x, edge_index, edge_attr, batch, token_index, W, bias):
    del batch
    g0, g1 = _node_tables(x, W, bias)
    sc = pl.kernel(
        _sc_body,
        out_type=jax.ShapeDtypeStruct((R, D), jnp.float32),
        mesh=plsc.VectorSubcoreMesh(
            core_axis_name="c", subcore_axis_name="s"
        ),
        scratch_types=_SC_SCRATCH,
        compiler_params=pltpu.CompilerParams(needs_layout_passes=False),
    )
    out = sc(
        edge_index[0], edge_index[1],
        token_index[0], token_index[1],
        edge_attr, g0, g1,
        jnp.zeros((TPT, D), jnp.float32),
    )
    return out.reshape(NG, NPG, NPG, D)


# speedup vs baseline: 26.0022x; 1.0674x over previous
import jax
import jax.numpy as jnp
from jax import lax
from jax.experimental import pallas as pl
from jax.experimental.pallas import tpu as pltpu
from jax.experimental.pallas import tpu_sc as plsc

N = 10240
E = 163840
D = 128
NG = 256
NPG = 40
R = NG * NPG * NPG

SHIFT = 13
BUCKET = 1 << SHIFT
NBUCKET = R // BUCKET
NCORE = 2
NSUB = 16
NPASS = NBUCKET // NCORE
LANES = 16
EC = E // NSUB
NV = EC // LANES
CH = 112
PAD_ROWS = 8
TPT = BUCKET // NSUB
IDB = 14
IDM = (1 << IDB) - 1


def _mm_body(x_ref, w_ref, b_ref, g0_ref, g1_ref):
    xb = x_ref[...]
    w = w_ref[...]
    g0_ref[...] = (
        jnp.dot(xb, w[:D, :], preferred_element_type=jnp.float32) + b_ref[...]
    )
    g1_ref[...] = jnp.dot(xb, w[D:, :], preferred_element_type=jnp.float32)


def _node_tables(x, w, bias):
    bm = 2048
    return pl.pallas_call(
        _mm_body,
        grid=(N // bm,),
        in_specs=[
            pl.BlockSpec((bm, D), lambda i: (i, 0)),
            pl.BlockSpec((2 * D, D), lambda i: (0, 0)),
            pl.BlockSpec((1, D), lambda i: (0, 0)),
        ],
        out_specs=[
            pl.BlockSpec((bm, D), lambda i: (i, 0)),
            pl.BlockSpec((bm, D), lambda i: (i, 0)),
        ],
        out_shape=[
            jax.ShapeDtypeStruct((N, D), jnp.float32),
            jax.ShapeDtypeStruct((N, D), jnp.float32),
        ],
    )(x, w, bias.reshape(1, D))


def _lane(v, j):
    return jnp.max(jnp.where(lax.iota(jnp.int32, LANES) == j, v, 0))


def _mod40(v):
    return v - ((v * 52429) >> 21) * NPG


def _cdiv_ch(n):
    return (((n + CH - 1) >> 4) * 9363) >> 16


def _sc_body(src_h, dst_h, t0_h, t1_h, ea_h, g0_h, g1_h, zsrc_h, out_h,
             buf1, buf2, buf3, off_a, off_b,
             gx0, gx1, rx0, rx1, rb0, rb1, acc,
             sem_g0, sem_g1, sem_s0, sem_s1):
    c = lax.axis_index("c")
    s = lax.axis_index("s")
    base_e = s * EC
    iota = lax.iota(jnp.int32, LANES)
    zeros16 = jnp.zeros((LANES,), jnp.int32)

    my_row0 = s * TPT
    pltpu.sync_copy(zsrc_h, acc.at[pl.ds(my_row0, TPT)])

    for k in range(NBUCKET):
        off_a[pl.ds(k * LANES, LANES)] = zeros16
        off_b[pl.ds(k * LANES, LANES)] = zeros16

    pltpu.sync_copy(src_h.at[pl.ds(base_e, EC)], buf1)
    pltpu.sync_copy(dst_h.at[pl.ds(base_e, EC)], buf3)

    def histo_e(i, _):
        src16 = buf1[pl.ds(i * LANES, LANES)]
        dst16 = buf3[pl.ds(i * LANES, LANES)]
        fe16 = src16 * NPG + _mod40(dst16)
        buf1[pl.ds(i * LANES, LANES)] = fe16
        sla = ((fe16 >> SHIFT) << 4) + iota
        plsc.store_scatter(off_a, [sla], plsc.load_gather(off_a, [sla]) + 1)
        return 0

    lax.fori_loop(0, NV, histo_e, 0)

    pltpu.sync_copy(t0_h.at[pl.ds(base_e, EC)], buf2)
    pltpu.sync_copy(t1_h.at[pl.ds(base_e, EC)], buf3)

    def histo_t(i, _):
        t016 = buf2[pl.ds(i * LANES, LANES)]
        t116 = buf3[pl.ds(i * LANES, LANES)]
        ft16 = t016 * NPG + _mod40(t116)
        buf2[pl.ds(i * LANES, LANES)] = t016 + (t116 << IDB)
        slb = ((ft16 >> SHIFT) << 4) + iota
        plsc.store_scatter(off_b, [slb], plsc.load_gather(off_b, [slb]) + 1)
        return 0

    lax.fori_loop(0, NV, histo_t, 0)

    def prefix(off_ref):
        def pbody(k, carry):
            v = off_ref[pl.ds(k * LANES, LANES)]
            incl = plsc.cumsum(v)
            off_ref[pl.ds(k * LANES, LANES)] = incl - v + carry
            return carry + jnp.max(incl)

        lax.fori_loop(0, NBUCKET, pbody, jnp.int32(0))

    prefix(off_a)
    prefix(off_b)

    def permute_e(i, _):
        id16 = i * LANES + iota
        fe16 = buf1[pl.ds(i * LANES, LANES)]
        sla = ((fe16 >> SHIFT) << 4) + iota
        pa = plsc.load_gather(off_a, [sla])
        plsc.store_scatter(buf3, [pa], ((fe16 & (BUCKET - 1)) << IDB) + id16)
        plsc.store_scatter(off_a, [sla], pa + 1)
        return 0

    lax.fori_loop(0, NV, permute_e, 0)

    def permute_t(i, _):
        id16 = i * LANES + iota
        t01 = buf2[pl.ds(i * LANES, LANES)]
        ft16 = (t01 & IDM) * NPG + _mod40(t01 >> IDB)
        slb = ((ft16 >> SHIFT) << 4) + iota
        pb = plsc.load_gather(off_b, [slb])
        plsc.store_scatter(buf1, [pb], ((ft16 & (BUCKET - 1)) << IDB) + id16)
        plsc.store_scatter(off_b, [slb], pb + 1)
        return 0

    lax.fori_loop(0, NV, permute_t, 0)

    plsc.subcore_barrier()

    pad_rows = BUCKET + (iota & (PAD_ROWS - 1))
    pad_nodes = s * LANES + iota

    def bucket_bounds(off_ref, b):
        bm1 = jnp.maximum(b - 1, 0)
        sa = jnp.where(
            b == 0, 0, _lane(off_ref[pl.ds(bm1 * LANES, LANES)], 15))
        ea = _lane(off_ref[pl.ds(b * LANES, LANES)], 15)
        return sa, ea

    def pass_body(p, _):
        b = NCORE * p + c
        base_row = b * BUCKET
        sa, ea = bucket_bounds(off_a, b)
        sb, eb = bucket_bounds(off_b, b)
        ca = _cdiv_ch(ea - sa)
        cb = _cdiv_ch(eb - sb)
        ntot = ca + 2 * cb

        def build_fire(t, gx, rx, rb, sem):
            @pl.when(t < ca)
            def _():
                st = sa + t * CH
                for k in range(CH // LANES):
                    n16 = st + k * LANES + iota
                    valid = n16 < ea
                    pk = plsc.load_gather(
                        buf3, [jnp.where(valid, n16, sa)])
                    gx[pl.ds(k * LANES, LANES)] = jnp.where(
                        valid, (pk & IDM) + base_e, pad_nodes)
                    rx[pl.ds(k * LANES, LANES)] = jnp.where(
                        valid, pk >> IDB, pad_rows)
                pltpu.async_copy(ea_h.at[gx], rb, sem)

            @pl.when(t >= ca)
            def _():
                tb = t - ca
                ci = jnp.where(tb < cb, tb, tb - cb)
                is0 = tb < cb
                st = sb + ci * CH
                for k in range(CH // LANES):
                    n16 = st + k * LANES + iota
                    valid = n16 < eb
                    pk = plsc.load_gather(
                        buf1, [jnp.where(valid, n16, sb)])
                    t01 = plsc.load_gather(buf2, [pk & IDM])
                    t_sel = jnp.where(is0, t01 & IDM, t01 >> IDB)
                    gx[pl.ds(k * LANES, LANES)] = jnp.where(
                        valid, t_sel, pad_nodes)
                    rx[pl.ds(k * LANES, LANES)] = jnp.where(
                        valid, pk >> IDB, pad_rows)

                @pl.when(is0)
                def _():
                    pltpu.async_copy(g0_h.at[gx], rb, sem)

                @pl.when(jnp.logical_not(is0))
                def _():
                    pltpu.async_copy(g1_h.at[gx], rb, sem)

        def pair_body(q, _):
            t0 = 2 * q
            t1 = t0 + 1
            build_fire(t0, gx0, rx0, rb0, sem_g0)

            @pl.when(t1 < ntot)
            def _():
                build_fire(t1, gx1, rx1, rb1, sem_g1)

            pltpu.make_async_copy(ea_h.at[gx0], rb0, sem_g0).wait()
            pltpu.async_copy(rb0, acc.at[rx0], sem_s0, add=True)

            @pl.when(t1 < ntot)
            def _():
                pltpu.make_async_copy(ea_h.at[gx1], rb1, sem_g1).wait()
                pltpu.async_copy(rb1, acc.at[rx1], sem_s1, add=True)

            pltpu.make_async_copy(rb0, acc.at[rx0], sem_s0).wait()

            @pl.when(t1 < ntot)
            def _():
                pltpu.make_async_copy(rb1, acc.at[rx1], sem_s1).wait()

            return 0

        lax.fori_loop(0, (ntot + 1) >> 1, pair_body, 0)

        plsc.subcore_barrier()
        pltpu.sync_copy(
            acc.at[pl.ds(my_row0, TPT)],
            out_h.at[pl.ds(base_row + my_row0, TPT)],
        )

        @pl.when(p < NPASS - 1)
        def _():
            pltpu.sync_copy(zsrc_h, acc.at[pl.ds(my_row0, TPT)])

        plsc.subcore_barrier()
        return 0

    lax.fori_loop(0, NPASS, pass_body, 0)


_SC_SCRATCH = [
    pltpu.VMEM((EC,), jnp.int32),
    pltpu.VMEM((EC,), jnp.int32),
    pltpu.VMEM((EC,), jnp.int32),
    pltpu.VMEM((NBUCKET * LANES,), jnp.int32),
    pltpu.VMEM((NBUCKET * LANES,), jnp.int32),
    pltpu.VMEM((CH,), jnp.int32),
    pltpu.VMEM((CH,), jnp.int32),
    pltpu.VMEM((CH,), jnp.int32),
    pltpu.VMEM((CH,), jnp.int32),
    pltpu.VMEM((CH, D), jnp.float32),
    pltpu.VMEM((CH, D), jnp.float32),
    pltpu.VMEM_SHARED((BUCKET + PAD_ROWS, D), jnp.float32),
    pltpu.SemaphoreType.DMA,
    pltpu.SemaphoreType.DMA,
    pltpu.SemaphoreType.DMA,
    pltpu.SemaphoreType.DMA,
]


@jax.jit
def kernel(x, edge_index, edge_attr, batch, token_index, W, bias):
    del batch
    g0, g1 = _node_tables(x, W, bias)
    sc = pl.kernel(
        _sc_body,
        out_type=jax.ShapeDtypeStruct((R, D), jnp.float32),
        mesh=plsc.VectorSubcoreMesh(
            core_axis_name="c", subcore_axis_name="s"
        ),
        scratch_types=_SC_SCRATCH,
        compiler_params=pltpu.CompilerParams(needs_layout_passes=False),
    )
    out = sc(
        edge_index[0], edge_index[1],
        token_index[0], token_index[1],
        edge_attr, g0, g1,
        jnp.zeros((TPT, D), jnp.float32),
    )
    return out.reshape(NG, NPG, NPG, D)


# speedup vs baseline: 27.2117x; 1.0465x over previous
import jax
import jax.numpy as jnp
from jax import lax
from jax.experimental import pallas as pl
from jax.experimental.pallas import tpu as pltpu
from jax.experimental.pallas import tpu_sc as plsc

N = 10240
E = 163840
D = 128
NG = 256
NPG = 40
R = NG * NPG * NPG

SHIFT = 12
BUCKET = 1 << SHIFT
NBUCKET = R // BUCKET
NCORE = 2
NSUB = 16
NPASS = NBUCKET // NCORE
LANES = 16
EC = E // NSUB
NV = EC // LANES
CH = 64
NBUF = 3
PAD_ROWS = 8
TPT = BUCKET // NSUB
IDB = 14
IDM = (1 << IDB) - 1


def _mm_body(x_ref, w_ref, b_ref, g0_ref, g1_ref):
    xb = x_ref[...]
    w = w_ref[...]
    g0_ref[...] = (
        jnp.dot(xb, w[:D, :], preferred_element_type=jnp.float32) + b_ref[...]
    )
    g1_ref[...] = jnp.dot(xb, w[D:, :], preferred_element_type=jnp.float32)


def _node_tables(x, w, bias):
    bm = 2048
    return pl.pallas_call(
        _mm_body,
        grid=(N // bm,),
        in_specs=[
            pl.BlockSpec((bm, D), lambda i: (i, 0)),
            pl.BlockSpec((2 * D, D), lambda i: (0, 0)),
            pl.BlockSpec((1, D), lambda i: (0, 0)),
        ],
        out_specs=[
            pl.BlockSpec((bm, D), lambda i: (i, 0)),
            pl.BlockSpec((bm, D), lambda i: (i, 0)),
        ],
        out_shape=[
            jax.ShapeDtypeStruct((N, D), jnp.float32),
            jax.ShapeDtypeStruct((N, D), jnp.float32),
        ],
    )(x, w, bias.reshape(1, D))


def _lane(v, j):
    return jnp.max(jnp.where(lax.iota(jnp.int32, LANES) == j, v, 0))


def _mod40(v):
    return v - ((v * 52429) >> 21) * NPG


def _sc_body(src_h, dst_h, t0_h, t1_h, ea_h, g0_h, g1_h, zsrc_h, out_h,
             buf1, buf2, buf3, off_a, off_b,
             gx0, gx1, gx2, rx0, rx1, rx2, rb0, rb1, rb2, acc,
             sg0, sg1, sg2, ss0, ss1, ss2, sc0, sc1, sz0, sz1):
    c = lax.axis_index("c")
    s = lax.axis_index("s")
    base_e = s * EC
    iota = lax.iota(jnp.int32, LANES)
    zeros16 = jnp.zeros((LANES,), jnp.int32)
    gx = (gx0, gx1, gx2)
    rx = (rx0, rx1, rx2)
    rb = (rb0, rb1, rb2)
    sg = (sg0, sg1, sg2)
    ss = (ss0, ss1, ss2)
    scp = (sc0, sc1)
    szp = (sz0, sz1)

    my_row0 = s * TPT
    pltpu.sync_copy(zsrc_h, acc.at[0, pl.ds(my_row0, TPT)])
    pltpu.sync_copy(zsrc_h, acc.at[1, pl.ds(my_row0, TPT)])

    for k in range(NBUCKET):
        off_a[pl.ds(k * LANES, LANES)] = zeros16
        off_b[pl.ds(k * LANES, LANES)] = zeros16

    pltpu.sync_copy(src_h.at[pl.ds(base_e, EC)], buf1)
    pltpu.sync_copy(dst_h.at[pl.ds(base_e, EC)], buf3)

    def histo_e(i, _):
        src16 = buf1[pl.ds(i * LANES, LANES)]
        dst16 = buf3[pl.ds(i * LANES, LANES)]
        fe16 = src16 * NPG + _mod40(dst16)
        buf1[pl.ds(i * LANES, LANES)] = fe16
        sla = ((fe16 >> SHIFT) << 4) + iota
        plsc.store_scatter(off_a, [sla], plsc.load_gather(off_a, [sla]) + 1)
        return 0

    lax.fori_loop(0, NV, histo_e, 0)

    pltpu.sync_copy(t0_h.at[pl.ds(base_e, EC)], buf2)
    pltpu.sync_copy(t1_h.at[pl.ds(base_e, EC)], buf3)

    def histo_t(i, _):
        t016 = buf2[pl.ds(i * LANES, LANES)]
        t116 = buf3[pl.ds(i * LANES, LANES)]
        ft16 = t016 * NPG + _mod40(t116)
        buf2[pl.ds(i * LANES, LANES)] = t016 + (t116 << IDB)
        slb = ((ft16 >> SHIFT) << 4) + iota
        plsc.store_scatter(off_b, [slb], plsc.load_gather(off_b, [slb]) + 1)
        return 0

    lax.fori_loop(0, NV, histo_t, 0)

    def prefix(off_ref):
        def pbody(k, carry):
            v = off_ref[pl.ds(k * LANES, LANES)]
            incl = plsc.cumsum(v)
            off_ref[pl.ds(k * LANES, LANES)] = incl - v + carry
            return carry + jnp.max(incl)

        lax.fori_loop(0, NBUCKET, pbody, jnp.int32(0))

    prefix(off_a)
    prefix(off_b)

    def permute_e(i, _):
        id16 = i * LANES + iota
        fe16 = buf1[pl.ds(i * LANES, LANES)]
        sla = ((fe16 >> SHIFT) << 4) + iota
        pa = plsc.load_gather(off_a, [sla])
        plsc.store_scatter(buf3, [pa], ((fe16 & (BUCKET - 1)) << IDB) + id16)
        plsc.store_scatter(off_a, [sla], pa + 1)
        return 0

    lax.fori_loop(0, NV, permute_e, 0)

    def permute_t(i, _):
        id16 = i * LANES + iota
        t01 = buf2[pl.ds(i * LANES, LANES)]
        ft16 = (t01 & IDM) * NPG + _mod40(t01 >> IDB)
        slb = ((ft16 >> SHIFT) << 4) + iota
        pb = plsc.load_gather(off_b, [slb])
        plsc.store_scatter(buf1, [pb], ((ft16 & (BUCKET - 1)) << IDB) + id16)
        plsc.store_scatter(off_b, [slb], pb + 1)
        return 0

    lax.fori_loop(0, NV, permute_t, 0)

    plsc.subcore_barrier()

    pad_rows = BUCKET + (iota & (PAD_ROWS - 1))
    pad_nodes = s * LANES + iota

    def bucket_bounds(off_ref, b):
        bm1 = jnp.maximum(b - 1, 0)
        sa = jnp.where(
            b == 0, 0, _lane(off_ref[pl.ds(bm1 * LANES, LANES)], 15))
        ea = _lane(off_ref[pl.ds(b * LANES, LANES)], 15)
        return sa, ea

    def do_pass(p, a, first):
        b = NCORE * p + c
        base_row = b * BUCKET
        acc_a = acc.at[a]
        sa, ea = bucket_bounds(off_a, b)
        sb, eb = bucket_bounds(off_b, b)
        ca = (ea - sa + CH - 1) >> 6
        cb = (eb - sb + CH - 1) >> 6
        ntot = ca + 2 * cb

        if not first:
            pltpu.make_async_copy(
                acc.at[1 - a, pl.ds(my_row0, TPT)],
                out_h.at[pl.ds(my_row0, TPT)],
                scp[1 - a],
            ).wait()
            pltpu.async_copy(
                zsrc_h, acc.at[1 - a, pl.ds(my_row0, TPT)], szp[1 - a])

        @pl.when(p >= 2)
        def _():
            pltpu.make_async_copy(
                zsrc_h, acc.at[a, pl.ds(my_row0, TPT)], szp[a]).wait()

        plsc.subcore_barrier()

        def build_fire(t, j):
            @pl.when(t < ca)
            def _():
                st = sa + t * CH
                for k in range(CH // LANES):
                    n16 = st + k * LANES + iota
                    valid = n16 < ea
                    pk = plsc.load_gather(
                        buf3, [jnp.where(valid, n16, sa)])
                    gx[j][pl.ds(k * LANES, LANES)] = jnp.where(
                        valid, (pk & IDM) + base_e, pad_nodes)
                    rx[j][pl.ds(k * LANES, LANES)] = jnp.where(
                        valid, pk >> IDB, pad_rows)
                pltpu.async_copy(ea_h.at[gx[j]], rb[j], sg[j])

            @pl.when(t >= ca)
            def _():
                tb = t - ca
                is0 = tb < cb
                ci = jnp.where(is0, tb, tb - cb)
                st = sb + ci * CH
                for k in range(CH // LANES):
                    n16 = st + k * LANES + iota
                    valid = n16 < eb
                    pk = plsc.load_gather(
                        buf1, [jnp.where(valid, n16, sb)])
                    t01 = plsc.load_gather(buf2, [pk & IDM])
                    t_sel = jnp.where(is0, t01 & IDM, t01 >> IDB)
                    gx[j][pl.ds(k * LANES, LANES)] = jnp.where(
                        valid, t_sel, pad_nodes)
                    rx[j][pl.ds(k * LANES, LANES)] = jnp.where(
                        valid, pk >> IDB, pad_rows)

                @pl.when(is0)
                def _():
                    pltpu.async_copy(g0_h.at[gx[j]], rb[j], sg[j])

                @pl.when(jnp.logical_not(is0))
                def _():
                    pltpu.async_copy(g1_h.at[gx[j]], rb[j], sg[j])

        def trip_body(q, _):
            t0 = NBUF * q
            for j in range(NBUF):
                t = t0 + j

                @pl.when(t < ntot)
                def _(t=t, j=j):
                    @pl.when(t >= NBUF)
                    def _():
                        pltpu.make_async_copy(
                            rb[j], acc_a.at[rx[j]], ss[j]).wait()

                    build_fire(t, j)

            for j in range(NBUF):
                t = t0 + j

                @pl.when(t < ntot)
                def _(t=t, j=j):
                    pltpu.make_async_copy(
                        ea_h.at[gx[j]], rb[j], sg[j]).wait()
                    pltpu.async_copy(rb[j], acc_a.at[rx[j]], ss[j], add=True)

            return 0

        lax.fori_loop(0, ((ntot + 2) * 10923) >> 15, trip_body, 0)

        for j in range(NBUF):
            @pl.when(ntot > j)
            def _(j=j):
                pltpu.make_async_copy(rb[j], acc_a.at[rx[j]], ss[j]).wait()

        plsc.subcore_barrier()
        pltpu.async_copy(
            acc.at[a, pl.ds(my_row0, TPT)],
            out_h.at[pl.ds(base_row + my_row0, TPT)],
            scp[a],
        )

    def pair_body(q, _):
        do_pass(2 * q, 0, False)
        do_pass(2 * q + 1, 1, False)
        return 0

    do_pass(jnp.int32(0), 0, True)
    do_pass(jnp.int32(1), 1, False)
    lax.fori_loop(1, NPASS // 2, pair_body, 0)

    pltpu.make_async_copy(
        acc.at[1, pl.ds(my_row0, TPT)],
        out_h.at[pl.ds(my_row0, TPT)],
        scp[1],
    ).wait()
    pltpu.make_async_copy(
        zsrc_h, acc.at[0, pl.ds(my_row0, TPT)], szp[0]).wait()


_SC_SCRATCH = [
    pltpu.VMEM((EC,), jnp.int32),
    pltpu.VMEM((EC,), jnp.int32),
    pltpu.VMEM((EC,), jnp.int32),
    pltpu.VMEM((NBUCKET * LANES,), jnp.int32),
    pltpu.VMEM((NBUCKET * LANES,), jnp.int32),
    pltpu.VMEM((CH,), jnp.int32),
    pltpu.VMEM((CH,), jnp.int32),
    pltpu.VMEM((CH,), jnp.int32),
    pltpu.VMEM((CH,), jnp.int32),
    pltpu.VMEM((CH,), jnp.int32),
    pltpu.VMEM((CH,), jnp.int32),
    pltpu.VMEM((CH, D), jnp.float32),
    pltpu.VMEM((CH, D), jnp.float32),
    pltpu.VMEM((CH, D), jnp.float32),
    pltpu.VMEM_SHARED((2, BUCKET + PAD_ROWS, D), jnp.float32),
    pltpu.SemaphoreType.DMA,
    pltpu.SemaphoreType.DMA,
    pltpu.SemaphoreType.DMA,
    pltpu.SemaphoreType.DMA,
    pltpu.SemaphoreType.DMA,
    pltpu.SemaphoreType.DMA,
    pltpu.SemaphoreType.DMA,
    pltpu.SemaphoreType.DMA,
    pltpu.SemaphoreType.DMA,
    pltpu.SemaphoreType.DMA,
]


@jax.jit
def kernel(x, edge_index, edge_attr, batch, token_index, W, bias):
    del batch
    g0, g1 = _node_tables(x, W, bias)
    sc = pl.kernel(
        _sc_body,
        out_type=jax.ShapeDtypeStruct((R, D), jnp.float32),
        mesh=plsc.VectorSubcoreMesh(
            core_axis_name="c", subcore_axis_name="s"
        ),
        scratch_types=_SC_SCRATCH,
        compiler_params=pltpu.CompilerParams(needs_layout_passes=False),
    )
    out = sc(
        edge_index[0], edge_index[1],
        token_index[0], token_index[1],
        edge_attr, g0, g1,
        jnp.zeros((TPT, D), jnp.float32),
    )
    return out.reshape(NG, NPG, NPG, D)
